# baseline (speedup 1.0000x reference)
import functools

import jax
import jax.numpy as jnp
from jax import lax
from jax.experimental import pallas as pl
from jax.experimental.pallas import tpu as pltpu
from jax.experimental.pallas import tpu_sc as plsc

D_MODEL = 64
VOCAB = 1000000
BATCH = 4096
HIST = 50

NC = 2
NS = 16
NW = NC * NS

B_TOTAL = BATCH * HIST
B_PER_W = B_TOTAL // NW
GRP = 128
N_GRP = B_PER_W // GRP
GRP_PER_STAGE = 5
N_STAGE = N_GRP // GRP_PER_STAGE
ROWS_PER_STAGE = GRP * GRP_PER_STAGE

SCALE = 8.0


def _mesh():
    return plsc.VectorSubcoreMesh(core_axis_name="c", subcore_axis_name="s")


@functools.partial(
    pl.kernel,
    mesh=_mesh(),
    out_type=jax.ShapeDtypeStruct((NW, N_STAGE, ROWS_PER_STAGE, D_MODEL),
                                  jnp.float32),
    scratch_types=[
        pltpu.VMEM((N_GRP, GRP), jnp.int32),
        pltpu.VMEM((2, ROWS_PER_STAGE, D_MODEL), jnp.float32),
        pltpu.SemaphoreType.DMA,
    ],
    compiler_params=pltpu.CompilerParams(use_tc_tiling_on_sc=False),
)
def _gather_scale(idx_hbm, table_hbm, out_hbm, idx_v, buf, sem):
    wid = lax.axis_index("s") * NC + lax.axis_index("c")
    pltpu.sync_copy(idx_hbm.at[wid], idx_v)

    for st in range(N_STAGE):
        p = st % 2
        descs = []
        for j in range(GRP_PER_STAGE):
            g = st * GRP_PER_STAGE + j
            descs.append(
                pltpu.async_copy(
                    table_hbm.at[idx_v.at[g]],
                    buf.at[p, pl.ds(j * GRP, GRP)],
                    sem,
                ))
        for d in descs:
            d.wait()

        def mul_body(r, carry):
            for q in range(D_MODEL // 16):
                sl = pl.ds(q * 16, 16)
                buf[p, r, sl] = buf[p, r, sl] * SCALE
            return carry

        lax.fori_loop(0, ROWS_PER_STAGE, mul_body, 0)
        pltpu.sync_copy(buf.at[p], out_hbm.at[wid, st])


def kernel(x, W):
    idx = x.reshape(NW, N_GRP, GRP).astype(jnp.int32)
    out = _gather_scale(idx, W)
    return out.reshape(BATCH, HIST, D_MODEL)


# speedup vs baseline: 1.0151x; 1.0151x over previous
import functools

import jax
import jax.numpy as jnp
from jax import lax
from jax.experimental import pallas as pl
from jax.experimental.pallas import tpu as pltpu
from jax.experimental.pallas import tpu_sc as plsc

D_MODEL = 64
VOCAB = 1000000
BATCH = 4096
HIST = 50

NC = 2
NS = 16
NW = NC * NS

B_TOTAL = BATCH * HIST
B_PER_W = B_TOTAL // NW
GRP = 128
N_GRP = B_PER_W // GRP
GRP_PER_STAGE = 5
N_STAGE = N_GRP // GRP_PER_STAGE
ROWS_PER_STAGE = GRP * GRP_PER_STAGE

SCALE = 8.0


def _mesh():
    return plsc.VectorSubcoreMesh(core_axis_name="c", subcore_axis_name="s")


@functools.partial(
    pl.kernel,
    mesh=_mesh(),
    out_type=jax.ShapeDtypeStruct((NW, N_STAGE, ROWS_PER_STAGE, D_MODEL),
                                  jnp.float32),
    scratch_types=[
        pltpu.VMEM((N_GRP, GRP), jnp.int32),
        pltpu.VMEM((2, ROWS_PER_STAGE, D_MODEL), jnp.float32),
        pltpu.SemaphoreType.DMA,
    ],
    compiler_params=pltpu.CompilerParams(use_tc_tiling_on_sc=False),
)
def _gather_scale(idx_hbm, table_hbm, out_hbm, idx_v, buf, sem):
    wid = lax.axis_index("s") * NC + lax.axis_index("c")
    pltpu.sync_copy(idx_hbm.at[wid], idx_v)

    for st in range(N_STAGE):
        p = st % 2
        descs = []
        for j in range(GRP_PER_STAGE):
            g = st * GRP_PER_STAGE + j
            descs.append(
                pltpu.async_copy(
                    table_hbm.at[idx_v.at[g]],
                    buf.at[p, pl.ds(j * GRP, GRP)],
                    sem,
                ))
        for d in descs:
            d.wait()

        def mul_body(r, carry):
            for q in range(D_MODEL // 16):
                sl = pl.ds(q * 16, 16)
                buf[p, r, sl] = buf[p, r, sl] * SCALE
            return carry

        lax.fori_loop(0, ROWS_PER_STAGE, mul_body, 0)
        pltpu.sync_copy(buf.at[p], out_hbm.at[wid, st])


def kernel(x, W):
    idx_t = jnp.transpose(x.reshape(BATCH, HIST)).astype(jnp.int32)
    idx = idx_t.reshape(NW, N_GRP, GRP)
    out = _gather_scale(idx, W)
    return jnp.transpose(out.reshape(HIST, BATCH, D_MODEL), (1, 0, 2))
